# baseline (speedup 1.0000x reference)
import functools

import jax
import jax.numpy as jnp
from jax import lax
from jax.experimental import pallas as pl
from jax.experimental.pallas import tpu as pltpu
from jax.experimental.pallas import tpu_sc as plsc

NN = 10000
NP = 10240
D = 128
E2 = 330000
CH = 96
NW = 32
CPW = 108
PAIRS = CPW // 2
PER_W = CPW * CH
EP = NW * PER_W
NPT = NP // 16
BLK = 1024
GRID = NP // BLK


def _pre_body(x_ref, w1_ref, b1_ref, gw_ref, asrc_ref, adst_ref,
              x1_ref, h_ref, sc_ref):
    f32 = jnp.float32
    x1 = jnp.dot(x_ref[...], w1_ref[...].T, preferred_element_type=f32)
    x1 = x1 + b1_ref[...]
    x1_ref[...] = x1
    rows = []
    for l in range(3):
        h_ref[l] = jnp.dot(x1, gw_ref[l].T, preferred_element_type=f32)
    for l in range(3):
        rows.append(jnp.dot(asrc_ref[l:l + 1, :], gw_ref[l],
                            preferred_element_type=f32))
    for l in range(3):
        rows.append(jnp.dot(adst_ref[l:l + 1, :], gw_ref[l],
                            preferred_element_type=f32))
    rows.append(jnp.zeros((2, D), f32))
    a_mat = jnp.concatenate(rows, axis=0)
    sc_ref[...] = lax.dot_general(
        x1, a_mat, dimension_numbers=(((1,), (1,)), ((), ())),
        preferred_element_type=f32)


def _pre_call(xp, W1, b1r, gat_W, att_src, att_dst):
    f32 = jnp.float32
    return pl.pallas_call(
        _pre_body,
        grid=(GRID,),
        in_specs=[
            pl.BlockSpec((BLK, D), lambda i: (i, 0)),
            pl.BlockSpec((D, D), lambda i: (0, 0)),
            pl.BlockSpec((1, D), lambda i: (0, 0)),
            pl.BlockSpec((3, D, D), lambda i: (0, 0, 0)),
            pl.BlockSpec((3, D), lambda i: (0, 0)),
            pl.BlockSpec((3, D), lambda i: (0, 0)),
        ],
        out_specs=[
            pl.BlockSpec((BLK, D), lambda i: (i, 0)),
            pl.BlockSpec((3, BLK, D), lambda i: (0, i, 0)),
            pl.BlockSpec((BLK, 8), lambda i: (i, 0)),
        ],
        out_shape=[
            jax.ShapeDtypeStruct((NP, D), f32),
            jax.ShapeDtypeStruct((3, NP, D), f32),
            jax.ShapeDtypeStruct((NP, 8), f32),
        ],
    )(xp, W1, b1r, gat_W, att_src, att_dst)


def _edge_body(h_hbm, sc_hbm, ed_hbm, z_hbm, z1_hbm,
               num_out, den_out,
               num_sh, den_sh, as_v, ad_v,
               sd0, sd1, exb0, exb1, sadj0, sadj1, dstb0, dstb1,
               rows0, rows1, isem0, isem1, gsem0, gsem1, ssem0, ssem1):
    f32 = jnp.float32
    c = lax.axis_index("c")
    s = lax.axis_index("s")
    wid = s * 2 + c
    iota = lax.iota(jnp.int32, 16)
    cbase = wid * CPW
    sd = (sd0, sd1)
    exb = (exb0, exb1)
    sadj = (sadj0, sadj1)
    dstb = (dstb0, dstb1)
    rows = (rows0, rows1)
    isem = (isem0, isem1)
    gsem = (gsem0, gsem1)
    ssem = (ssem0, ssem1)

    def fetch_idx(k, b):
        pltpu.async_copy(ed_hbm.at[cbase + k], sd[b], isem[b])

    def wait_idx(b):
        pltpu.make_async_copy(ed_hbm.at[0], sd[b], isem[b]).wait()

    def gather_rows(b):
        pltpu.async_copy(h_hbm.at[sadj[b]], rows[b], gsem[b])

    def wait_rows(b):
        pltpu.make_async_copy(h_hbm.at[sadj[b]], rows[b], gsem[b]).wait()

    def scatter_num(b):
        pltpu.async_copy(rows[b], num_sh.at[dstb[b]], ssem[b], add=True)

    def wait_scatter(b):
        pltpu.make_async_copy(rows[b], num_sh.at[dstb[b]], ssem[b]).wait()

    for l in range(3):
        pltpu.sync_copy(sc_hbm.at[l], as_v)
        pltpu.sync_copy(sc_hbm.at[3 + l], ad_v)

        @pl.when(s == 0)
        def _():
            pltpu.sync_copy(z_hbm, num_sh)
            pltpu.sync_copy(z1_hbm, den_sh)
        plsc.subcore_barrier()

        def vec_phase(k, b):
            def _vec(r, c2):
                si = sd[b][0, pl.ds(r * 16, 16)]
                di = sd[b][1, pl.ds(r * 16, 16)]
                sv = plsc.load_gather(as_v, [si])
                dv = plsc.load_gather(ad_v, [di])
                e = sv + dv
                e = jnp.where(e >= 0.0, e, 0.2 * e)
                gid = (cbase + k) * CH + r * 16 + iota
                ex = jnp.where(gid < E2, jnp.exp(e), 0.0)
                exb[b][pl.ds(r * 16, 16)] = ex
                sadj[b][pl.ds(r * 16, 16)] = si + l * NP
                dstb[b][pl.ds(r * 16, 16)] = di
                return c2
            lax.fori_loop(0, CH // 16, _vec, 0)

        def scale_scatter(b):
            pltpu.sync_copy(exb[b], den_sh.at[dstb[b]], add=True)

            def _scale(q, c2):
                for u in range(4):
                    r = q * 4 + u
                    ridx = iota * 0 + r
                    ex16 = plsc.load_gather(exb[b], [ridx])
                    for j in range(8):
                        sl = pl.ds(j * 16, 16)
                        rows[b][r, sl] = rows[b][r, sl] * ex16
                return c2
            lax.fori_loop(0, CH // 4, _scale, 0)
            scatter_num(b)

        fetch_idx(0, 0)
        wait_idx(0)
        vec_phase(0, 0)
        gather_rows(0)
        fetch_idx(1, 1)

        def _pair(kk, carry):
            a = 2 * kk
            last = kk == PAIRS - 1
            wait_idx(1)
            vec_phase(a + 1, 1)

            @pl.when(kk > 0)
            def _():
                wait_scatter(1)
            gather_rows(1)

            @pl.when(jnp.logical_not(last))
            def _():
                fetch_idx(a + 2, 0)
            wait_rows(0)
            scale_scatter(0)

            @pl.when(jnp.logical_not(last))
            def _():
                wait_idx(0)
                vec_phase(a + 2, 0)
            wait_rows(1)
            scale_scatter(1)

            @pl.when(jnp.logical_not(last))
            def _():
                wait_scatter(0)
                gather_rows(0)
                fetch_idx(a + 3, 1)
            return carry
        lax.fori_loop(0, PAIRS, _pair, 0)

        wait_scatter(0)
        wait_scatter(1)
        plsc.subcore_barrier()
        pltpu.sync_copy(den_sh.at[pl.ds(s * NPT, NPT)],
                        den_out.at[2 * l + c, pl.ds(s * NPT, NPT)])
        pltpu.sync_copy(num_sh.at[pl.ds(s * NPT, NPT)],
                        num_out.at[2 * l + c, pl.ds(s * NPT, NPT)])
        plsc.subcore_barrier()


def _edge_call(h_flat, scT, ed, zf, z1):
    f32 = jnp.float32
    i32 = jnp.int32
    mesh = plsc.VectorSubcoreMesh(core_axis_name="c", subcore_axis_name="s")
    fn = functools.partial(
        pl.kernel,
        out_type=[
            jax.ShapeDtypeStruct((6, NP, D), f32),
            jax.ShapeDtypeStruct((6, NP), f32),
        ],
        mesh=mesh,
        scratch_types=[
            pltpu.VMEM_SHARED((NP, D), f32),
            pltpu.VMEM_SHARED((NP,), f32),
            pltpu.VMEM((NP,), f32),
            pltpu.VMEM((NP,), f32),
            pltpu.VMEM((2, CH), i32),
            pltpu.VMEM((2, CH), i32),
            pltpu.VMEM((CH,), f32),
            pltpu.VMEM((CH,), f32),
            pltpu.VMEM((CH,), i32),
            pltpu.VMEM((CH,), i32),
            pltpu.VMEM((CH,), i32),
            pltpu.VMEM((CH,), i32),
            pltpu.VMEM((CH, D), f32),
            pltpu.VMEM((CH, D), f32),
            pltpu.SemaphoreType.DMA,
            pltpu.SemaphoreType.DMA,
            pltpu.SemaphoreType.DMA,
            pltpu.SemaphoreType.DMA,
            pltpu.SemaphoreType.DMA,
            pltpu.SemaphoreType.DMA,
        ],
        compiler_params=pltpu.CompilerParams(needs_layout_passes=False),
    )(_edge_body)
    return fn(h_flat, scT, ed, zf, z1)


def _post_body(num_ref, den_ref, x1_ref, gb_ref, wih_ref, whh_ref,
               w2_ref, b2_ref, o_ref):
    f32 = jnp.float32
    x = x1_ref[...]
    h = jnp.zeros((BLK, D), f32)
    cst = jnp.zeros((BLK, D), f32)
    for l in range(3):
        den = jnp.sum(den_ref[2 * l:2 * l + 2, :], axis=0) + 1e-16
        num = num_ref[2 * l] + num_ref[2 * l + 1]
        htmp = jnp.tanh(num / den[:, None] + gb_ref[l:l + 1, :])
        incat = jnp.concatenate([htmp, x], axis=-1)
        gates = (jnp.dot(incat, wih_ref[l].T, preferred_element_type=f32)
                 + jnp.dot(h, whh_ref[l].T, preferred_element_type=f32))
        i_ = jax.nn.sigmoid(gates[:, 0 * D:1 * D])
        f_ = jax.nn.sigmoid(gates[:, 1 * D:2 * D])
        g_ = jnp.tanh(gates[:, 2 * D:3 * D])
        o_ = jax.nn.sigmoid(gates[:, 3 * D:4 * D])
        cst = f_ * cst + i_ * g_
        h = o_ * jnp.tanh(cst)
        x = h
    o_ref[...] = jnp.dot(x, w2_ref[...].T, preferred_element_type=f32) \
        + b2_ref[...]


def _post_call(num6, den, x1, gat_b, w_ih, w_hh, W2, b2r):
    f32 = jnp.float32
    return pl.pallas_call(
        _post_body,
        grid=(GRID,),
        in_specs=[
            pl.BlockSpec((6, BLK, D), lambda i: (0, i, 0)),
            pl.BlockSpec((6, BLK), lambda i: (0, i)),
            pl.BlockSpec((BLK, D), lambda i: (i, 0)),
            pl.BlockSpec((3, D), lambda i: (0, 0)),
            pl.BlockSpec((3, 4 * D, 2 * D), lambda i: (0, 0, 0)),
            pl.BlockSpec((3, 4 * D, D), lambda i: (0, 0, 0)),
            pl.BlockSpec((D, D), lambda i: (0, 0)),
            pl.BlockSpec((1, D), lambda i: (0, 0)),
        ],
        out_specs=pl.BlockSpec((BLK, D), lambda i: (i, 0)),
        out_shape=jax.ShapeDtypeStruct((NP, D), f32),
    )(num6, den, x1, gat_b, w_ih, w_hh, W2, b2r)


def kernel(x, edge_index, W1, b1, gat_W, att_src, att_dst, gat_b,
           w_ih, w_hh, W2, b2):
    f32 = jnp.float32
    xp = jnp.pad(x.astype(f32), ((0, NP - NN), (0, 0)))
    b1r = b1.reshape(1, D).astype(f32)
    b2r = b2.reshape(1, D).astype(f32)

    loop = jnp.arange(NN, dtype=jnp.int32)
    pad = jnp.zeros((EP - E2,), jnp.int32)
    srcp = jnp.concatenate([edge_index[0].astype(jnp.int32), loop, pad])
    dstp = jnp.concatenate([edge_index[1].astype(jnp.int32), loop, pad])
    ed = jnp.stack([srcp.reshape(EP // CH, CH),
                    dstp.reshape(EP // CH, CH)], axis=1)

    x1, h_all, scores = _pre_call(xp, W1, b1r, gat_W, att_src, att_dst)
    h_flat = h_all.reshape(3 * NP, D)
    scT = scores.T
    zf = jnp.zeros((NP, D), f32)
    z1 = jnp.zeros((NP,), f32)

    num6, den = _edge_call(h_flat, scT, ed, zf, z1)
    out = _post_call(num6, den, x1, gat_b, w_ih, w_hh, W2, b2r)
    return out[:NN]


# speedup vs baseline: 1.0451x; 1.0451x over previous
import functools

import jax
import jax.numpy as jnp
from jax import lax
from jax.experimental import pallas as pl
from jax.experimental.pallas import tpu as pltpu
from jax.experimental.pallas import tpu_sc as plsc

NN = 10000
NP = 10240
D = 128
E2 = 330000
CH = 96
NW = 32
CPW = 108
PAIRS = CPW // 2
PER_W = CPW * CH
EP = NW * PER_W
NPT = NP // 16
BLK = 1024
GRID = NP // BLK


def _pre_body(x_ref, w1_ref, b1_ref, gw_ref, asrc_ref, adst_ref,
              x1_ref, h_ref, sc_ref):
    f32 = jnp.float32
    x1 = jnp.dot(x_ref[...], w1_ref[...].T, preferred_element_type=f32)
    x1 = x1 + b1_ref[...]
    x1_ref[...] = x1
    rows = []
    for l in range(3):
        h_ref[l] = jnp.dot(x1, gw_ref[l].T, preferred_element_type=f32)
    for l in range(3):
        rows.append(jnp.dot(asrc_ref[l:l + 1, :], gw_ref[l],
                            preferred_element_type=f32))
    for l in range(3):
        rows.append(jnp.dot(adst_ref[l:l + 1, :], gw_ref[l],
                            preferred_element_type=f32))
    rows.append(jnp.zeros((2, D), f32))
    a_mat = jnp.concatenate(rows, axis=0)
    sc_ref[...] = lax.dot_general(
        x1, a_mat, dimension_numbers=(((1,), (1,)), ((), ())),
        preferred_element_type=f32)


def _pre_call(xp, W1, b1r, gat_W, att_src, att_dst):
    f32 = jnp.float32
    return pl.pallas_call(
        _pre_body,
        grid=(GRID,),
        in_specs=[
            pl.BlockSpec((BLK, D), lambda i: (i, 0)),
            pl.BlockSpec((D, D), lambda i: (0, 0)),
            pl.BlockSpec((1, D), lambda i: (0, 0)),
            pl.BlockSpec((3, D, D), lambda i: (0, 0, 0)),
            pl.BlockSpec((3, D), lambda i: (0, 0)),
            pl.BlockSpec((3, D), lambda i: (0, 0)),
        ],
        out_specs=[
            pl.BlockSpec((BLK, D), lambda i: (i, 0)),
            pl.BlockSpec((3, BLK, D), lambda i: (0, i, 0)),
            pl.BlockSpec((BLK, 8), lambda i: (i, 0)),
        ],
        out_shape=[
            jax.ShapeDtypeStruct((NP, D), f32),
            jax.ShapeDtypeStruct((3, NP, D), f32),
            jax.ShapeDtypeStruct((NP, 8), f32),
        ],
    )(xp, W1, b1r, gat_W, att_src, att_dst)


def _edge_body(h_hbm, sc_hbm, ed_hbm, z_hbm, z1_hbm,
               num_out, den_out,
               num_sh, den_sh, as_v, ad_v,
               sd0, sd1, exb0, exb1, sadj0, sadj1, dstb0, dstb1,
               rows0, rows1, isem0, isem1, gsem0, gsem1):
    f32 = jnp.float32
    c = lax.axis_index("c")
    s = lax.axis_index("s")
    wid = s * 2 + c
    iota = lax.iota(jnp.int32, 16)
    cbase = wid * CPW
    sd = (sd0, sd1)
    exb = (exb0, exb1)
    sadj = (sadj0, sadj1)
    dstb = (dstb0, dstb1)
    rows = (rows0, rows1)
    isem = (isem0, isem1)
    gsem = (gsem0, gsem1)

    def fetch_idx(k, b):
        pltpu.async_copy(ed_hbm.at[cbase + k], sd[b], isem[b])

    def wait_idx(b):
        pltpu.make_async_copy(ed_hbm.at[0], sd[b], isem[b]).wait()

    def gather_rows(b):
        pltpu.async_copy(h_hbm.at[sadj[b]], rows[b], gsem[b])

    def wait_rows(b):
        pltpu.make_async_copy(h_hbm.at[sadj[b]], rows[b], gsem[b]).wait()

    for l in range(3):
        pltpu.sync_copy(sc_hbm.at[l], as_v)
        pltpu.sync_copy(sc_hbm.at[3 + l], ad_v)

        @pl.when(s == 0)
        def _():
            pltpu.sync_copy(z_hbm, num_sh)
            pltpu.sync_copy(z1_hbm, den_sh)
        plsc.subcore_barrier()

        def vec_phase(k, b):
            def _vec(r, c2):
                si = sd[b][0, pl.ds(r * 16, 16)]
                di = sd[b][1, pl.ds(r * 16, 16)]
                sv = plsc.load_gather(as_v, [si])
                dv = plsc.load_gather(ad_v, [di])
                e = sv + dv
                e = jnp.where(e >= 0.0, e, 0.2 * e)
                gid = (cbase + k) * CH + r * 16 + iota
                ex = jnp.where(gid < E2, jnp.exp(e), 0.0)
                exb[b][pl.ds(r * 16, 16)] = ex
                sadj[b][pl.ds(r * 16, 16)] = si + l * NP
                dstb[b][pl.ds(r * 16, 16)] = di
                return c2
            lax.fori_loop(0, CH // 16, _vec, 0)

        def scale_scatter(b):
            pltpu.sync_copy(exb[b], den_sh.at[dstb[b]], add=True)

            def _scale(q, c2):
                for u in range(4):
                    r = q * 4 + u
                    ridx = iota * 0 + r
                    ex16 = plsc.load_gather(exb[b], [ridx])
                    for j in range(8):
                        sl = pl.ds(j * 16, 16)
                        rows[b][r, sl] = rows[b][r, sl] * ex16
                return c2
            lax.fori_loop(0, CH // 4, _scale, 0)
            pltpu.sync_copy(rows[b], num_sh.at[dstb[b]], add=True)

        fetch_idx(0, 0)
        wait_idx(0)
        vec_phase(0, 0)
        gather_rows(0)
        fetch_idx(1, 1)

        def _pair(kk, carry):
            a = 2 * kk
            last = kk == PAIRS - 1
            wait_idx(1)
            vec_phase(a + 1, 1)
            gather_rows(1)

            @pl.when(jnp.logical_not(last))
            def _():
                fetch_idx(a + 2, 0)
            wait_rows(0)
            scale_scatter(0)

            @pl.when(jnp.logical_not(last))
            def _():
                wait_idx(0)
                vec_phase(a + 2, 0)
                gather_rows(0)
                fetch_idx(a + 3, 1)
            wait_rows(1)
            scale_scatter(1)
            return carry
        lax.fori_loop(0, PAIRS, _pair, 0)

        plsc.subcore_barrier()
        pltpu.sync_copy(den_sh.at[pl.ds(s * NPT, NPT)],
                        den_out.at[2 * l + c, pl.ds(s * NPT, NPT)])
        pltpu.sync_copy(num_sh.at[pl.ds(s * NPT, NPT)],
                        num_out.at[2 * l + c, pl.ds(s * NPT, NPT)])
        plsc.subcore_barrier()


def _edge_call(h_flat, scT, ed, zf, z1):
    f32 = jnp.float32
    i32 = jnp.int32
    mesh = plsc.VectorSubcoreMesh(core_axis_name="c", subcore_axis_name="s")
    fn = functools.partial(
        pl.kernel,
        out_type=[
            jax.ShapeDtypeStruct((6, NP, D), f32),
            jax.ShapeDtypeStruct((6, NP), f32),
        ],
        mesh=mesh,
        scratch_types=[
            pltpu.VMEM_SHARED((NP, D), f32),
            pltpu.VMEM_SHARED((NP,), f32),
            pltpu.VMEM((NP,), f32),
            pltpu.VMEM((NP,), f32),
            pltpu.VMEM((2, CH), i32),
            pltpu.VMEM((2, CH), i32),
            pltpu.VMEM((CH,), f32),
            pltpu.VMEM((CH,), f32),
            pltpu.VMEM((CH,), i32),
            pltpu.VMEM((CH,), i32),
            pltpu.VMEM((CH,), i32),
            pltpu.VMEM((CH,), i32),
            pltpu.VMEM((CH, D), f32),
            pltpu.VMEM((CH, D), f32),
            pltpu.SemaphoreType.DMA,
            pltpu.SemaphoreType.DMA,
            pltpu.SemaphoreType.DMA,
            pltpu.SemaphoreType.DMA,
        ],
        compiler_params=pltpu.CompilerParams(needs_layout_passes=False),
    )(_edge_body)
    return fn(h_flat, scT, ed, zf, z1)


def _post_body(num_ref, den_ref, x1_ref, gb_ref, wih_ref, whh_ref,
               w2_ref, b2_ref, o_ref):
    f32 = jnp.float32
    x = x1_ref[...]
    h = jnp.zeros((BLK, D), f32)
    cst = jnp.zeros((BLK, D), f32)
    for l in range(3):
        den = jnp.sum(den_ref[2 * l:2 * l + 2, :], axis=0) + 1e-16
        num = num_ref[2 * l] + num_ref[2 * l + 1]
        htmp = jnp.tanh(num / den[:, None] + gb_ref[l:l + 1, :])
        incat = jnp.concatenate([htmp, x], axis=-1)
        gates = (jnp.dot(incat, wih_ref[l].T, preferred_element_type=f32)
                 + jnp.dot(h, whh_ref[l].T, preferred_element_type=f32))
        i_ = jax.nn.sigmoid(gates[:, 0 * D:1 * D])
        f_ = jax.nn.sigmoid(gates[:, 1 * D:2 * D])
        g_ = jnp.tanh(gates[:, 2 * D:3 * D])
        o_ = jax.nn.sigmoid(gates[:, 3 * D:4 * D])
        cst = f_ * cst + i_ * g_
        h = o_ * jnp.tanh(cst)
        x = h
    o_ref[...] = jnp.dot(x, w2_ref[...].T, preferred_element_type=f32) \
        + b2_ref[...]


def _post_call(num6, den, x1, gat_b, w_ih, w_hh, W2, b2r):
    f32 = jnp.float32
    return pl.pallas_call(
        _post_body,
        grid=(GRID,),
        in_specs=[
            pl.BlockSpec((6, BLK, D), lambda i: (0, i, 0)),
            pl.BlockSpec((6, BLK), lambda i: (0, i)),
            pl.BlockSpec((BLK, D), lambda i: (i, 0)),
            pl.BlockSpec((3, D), lambda i: (0, 0)),
            pl.BlockSpec((3, 4 * D, 2 * D), lambda i: (0, 0, 0)),
            pl.BlockSpec((3, 4 * D, D), lambda i: (0, 0, 0)),
            pl.BlockSpec((D, D), lambda i: (0, 0)),
            pl.BlockSpec((1, D), lambda i: (0, 0)),
        ],
        out_specs=pl.BlockSpec((BLK, D), lambda i: (i, 0)),
        out_shape=jax.ShapeDtypeStruct((NP, D), f32),
    )(num6, den, x1, gat_b, w_ih, w_hh, W2, b2r)


def kernel(x, edge_index, W1, b1, gat_W, att_src, att_dst, gat_b,
           w_ih, w_hh, W2, b2):
    f32 = jnp.float32
    xp = jnp.pad(x.astype(f32), ((0, NP - NN), (0, 0)))
    b1r = b1.reshape(1, D).astype(f32)
    b2r = b2.reshape(1, D).astype(f32)

    loop = jnp.arange(NN, dtype=jnp.int32)
    pad = jnp.zeros((EP - E2,), jnp.int32)
    srcp = jnp.concatenate([edge_index[0].astype(jnp.int32), loop, pad])
    dstp = jnp.concatenate([edge_index[1].astype(jnp.int32), loop, pad])
    ed = jnp.stack([srcp.reshape(EP // CH, CH),
                    dstp.reshape(EP // CH, CH)], axis=1)

    x1, h_all, scores = _pre_call(xp, W1, b1r, gat_W, att_src, att_dst)
    h_flat = h_all.reshape(3 * NP, D)
    scT = scores.T
    zf = jnp.zeros((NP, D), f32)
    z1 = jnp.zeros((NP,), f32)

    num6, den = _edge_call(h_flat, scT, ed, zf, z1)
    out = _post_call(num6, den, x1, gat_b, w_ih, w_hh, W2, b2r)
    return out[:NN]
